# baseline (speedup 1.0000x reference)
import functools

import jax
import jax.numpy as jnp
from jax import lax
from jax.experimental import pallas as pl
from jax.experimental.pallas import tpu as pltpu
from jax.experimental.pallas import tpu_sc as plsc

N = 10000
E = 320000
D = 128
R = 11

NC = 2
NS = 16
LANES = 16
NW = NC * NS

NB = 80
B = 128
EPW = NB * B
E_PAD = NW * EPW
N_PAD = 10240
RPT = N_PAD // NS


CH = 16
NCH = NB // CH


def _sc_agg_body(with_deg, table, srch, eth, dsth, *refs):
    if with_deg:
        (acc_out, deg_out, acc_sh, deg_sh, idx_cv, et_cv, dst_cv,
         rows_a, rows_b, dvec_v, ones_v, sem_a, sem_b) = refs
    else:
        (acc_out, acc_sh, idx_cv, et_cv, dst_cv,
         rows_a, rows_b, sem_a, sem_b) = refs
    c = lax.axis_index("c")
    s = lax.axis_index("s")

    zv = jnp.zeros((LANES,), jnp.float32)

    @pl.loop(0, B)
    def _zero_rows(i):
        for j in range(D // LANES):
            rows_a[i, pl.ds(j * LANES, LANES)] = zv

    if with_deg:
        for j in range(RPT // LANES):
            dvec_v[pl.ds(j * LANES, LANES)] = zv
        for j in range(B // LANES):
            ones_v[pl.ds(j * LANES, LANES)] = jnp.ones((LANES,), jnp.float32)

    base = s * RPT
    for k in range(RPT // B):
        pltpu.sync_copy(rows_a, acc_sh.at[pl.ds(base + k * B, B)])
    if with_deg:
        pltpu.sync_copy(dvec_v, deg_sh.at[pl.ds(base, RPT)])

    plsc.subcore_barrier()

    @pl.loop(0, NCH)
    def _chunk(ch):
        sl_ch = pl.ds(ch * CH, CH)
        pltpu.sync_copy(srch.at[c, s, sl_ch], idx_cv)
        pltpu.sync_copy(eth.at[c, s, sl_ch], et_cv)
        pltpu.sync_copy(dsth.at[c, s, sl_ch], dst_cv)

        for i in range(CH):
            for j in range(B // LANES):
                sl = pl.ds(j * LANES, LANES)
                idx_cv[i, sl] = et_cv[i, sl] * N_PAD + idx_cv[i, sl]

        pltpu.async_copy(table.at[idx_cv.at[0]], rows_a, sem_a)
        pltpu.async_copy(table.at[idx_cv.at[1]], rows_b, sem_b)
        for j in range(2, CH + 2):
            buf, sem = (rows_a, sem_a) if j % 2 == 0 else (rows_b, sem_b)
            pltpu.make_async_copy(table.at[idx_cv.at[j - 2]], buf, sem).wait()
            pltpu.sync_copy(buf, acc_sh.at[dst_cv.at[j - 2]], add=True)
            if j < CH:
                pltpu.async_copy(table.at[idx_cv.at[j]], buf, sem)
            if with_deg:
                pltpu.sync_copy(ones_v, deg_sh.at[dst_cv.at[j - 2]], add=True)

    plsc.subcore_barrier()

    for k in range(RPT // B):
        pltpu.sync_copy(acc_sh.at[pl.ds(base + k * B, B)], rows_a)
        pltpu.sync_copy(rows_a, acc_out.at[c, pl.ds(base + k * B, B)])
    if with_deg:
        pltpu.sync_copy(deg_sh.at[pl.ds(base, RPT)], dvec_v)
        pltpu.sync_copy(dvec_v, deg_out.at[c, pl.ds(base, RPT)])


@functools.lru_cache(maxsize=None)
def _sc_agg_kernel(with_deg):
    out_type = [jax.ShapeDtypeStruct((NC, N_PAD, D), jnp.float32)]
    scratch = [pltpu.VMEM_SHARED((N_PAD, D), jnp.float32)]
    if with_deg:
        out_type.append(jax.ShapeDtypeStruct((NC, N_PAD), jnp.float32))
        scratch.append(pltpu.VMEM_SHARED((N_PAD,), jnp.float32))
    scratch += [
        pltpu.VMEM((CH, B), jnp.int32),
        pltpu.VMEM((CH, B), jnp.int32),
        pltpu.VMEM((CH, B), jnp.int32),
        pltpu.VMEM((B, D), jnp.float32),
        pltpu.VMEM((B, D), jnp.float32),
    ]
    if with_deg:
        scratch += [
            pltpu.VMEM((RPT,), jnp.float32),
            pltpu.VMEM((B,), jnp.float32),
        ]
    scratch += [pltpu.SemaphoreType.DMA, pltpu.SemaphoreType.DMA]
    return pl.kernel(
        functools.partial(_sc_agg_body, with_deg),
        out_type=out_type,
        mesh=plsc.VectorSubcoreMesh(
            core_axis_name="c", subcore_axis_name="s",
            num_cores=NC, num_subcores=NS),
        scratch_types=scratch,
    )


_CB = 1024


def _scale_body(x_ref, rw_ref, o_ref):
    o_ref[...] = x_ref[...][None] * rw_ref[...][:, None, :]


def _build_table(x_pad, relw):
    out = pl.pallas_call(
        _scale_body,
        grid=(N_PAD // _CB,),
        in_specs=[
            pl.BlockSpec((_CB, D), lambda i: (i, 0)),
            pl.BlockSpec((R, D), lambda i: (0, 0)),
        ],
        out_specs=pl.BlockSpec((R, _CB, D), lambda i: (0, i, 0)),
        out_shape=jax.ShapeDtypeStruct((R, N_PAD, D), jnp.float32),
    )(x_pad, relw)
    return out.reshape(R * N_PAD, D)


def _combine_body(with_table, *refs):
    if with_table:
        acc_ref, deg_ref, x_ref, w_ref, ws_ref, b_ref, rw_ref, o_ref, t_ref = refs
    else:
        acc_ref, deg_ref, x_ref, w_ref, ws_ref, b_ref, o_ref = refs
    agg = acc_ref[0] + acc_ref[1]
    deg = deg_ref[:, 0] + deg_ref[:, 1]
    inv = 1.0 / jnp.maximum(deg, 1.0)
    agg = agg * inv[:, None]
    h = (jnp.dot(agg, w_ref[...], preferred_element_type=jnp.float32)
         + jnp.dot(x_ref[...], ws_ref[...], preferred_element_type=jnp.float32)
         + b_ref[...])
    h = jnp.maximum(h, 0.0)
    o_ref[...] = h
    if with_table:
        t_ref[...] = h[None] * rw_ref[...][:, None, :]


def _combine(acc, deg_t, x_pad, w, wself, b, relw_next=None):
    with_table = relw_next is not None
    rb = _CB if with_table else 1000
    n_out = N_PAD if with_table else N
    in_specs = [
        pl.BlockSpec((NC, rb, D), lambda i: (0, i, 0)),
        pl.BlockSpec((rb, NC), lambda i: (i, 0)),
        pl.BlockSpec((rb, D), lambda i: (i, 0)),
        pl.BlockSpec((D, D), lambda i: (0, 0)),
        pl.BlockSpec((D, D), lambda i: (0, 0)),
        pl.BlockSpec((1, D), lambda i: (0, 0)),
    ]
    out_specs = [pl.BlockSpec((rb, D), lambda i: (i, 0))]
    out_shape = [jax.ShapeDtypeStruct((n_out, D), jnp.float32)]
    args = [acc, deg_t, x_pad, w, wself, b]
    if with_table:
        in_specs.append(pl.BlockSpec((R, D), lambda i: (0, 0)))
        out_specs.append(pl.BlockSpec((R, rb, D), lambda i: (0, i, 0)))
        out_shape.append(jax.ShapeDtypeStruct((R, N_PAD, D), jnp.float32))
        args.append(relw_next)
    res = pl.pallas_call(
        functools.partial(_combine_body, with_table),
        grid=(n_out // rb,),
        in_specs=in_specs,
        out_specs=out_specs,
        out_shape=out_shape,
    )(*args)
    if with_table:
        return res[0], res[1].reshape(R * N_PAD, D)
    return res[0]


def kernel(node_init, W1, Wself1, b1, relw1, W2, Wself2, b2, relw2,
           edge_index, edge_type):
    src = edge_index[0].astype(jnp.int32)
    dst = edge_index[1].astype(jnp.int32)
    et = edge_type.astype(jnp.int32)

    pad = E_PAD - E
    pad_ar = jnp.arange(pad, dtype=jnp.int32)
    src_p = jnp.concatenate([src, (pad_ar * 97) % N])
    dst_p = jnp.concatenate([dst, N + (pad_ar % (N_PAD - N))])
    et_p = jnp.concatenate([et, pad_ar % R])
    shp = (NC, NS, NB, B)
    src_p = src_p.reshape(shp)
    dst_p = dst_p.reshape(shp)
    et_p = et_p.reshape(shp)

    b1r = b1.reshape(1, D)
    b2r = b2.reshape(1, D)
    x_pad = jnp.pad(node_init, ((0, N_PAD - N), (0, 0)))

    table1 = _build_table(x_pad, relw1)
    acc1, deg1 = _sc_agg_kernel(True)(table1, src_p, et_p, dst_p)
    deg_t = deg1.T
    h1p, table2 = _combine(acc1, deg_t, x_pad, W1, Wself1, b1r, relw2)

    (acc2,) = _sc_agg_kernel(False)(table2, src_p, et_p, dst_p)
    return _combine(acc2, deg_t, h1p, W2, Wself2, b2r)


# speedup vs baseline: 1.0738x; 1.0738x over previous
import functools

import jax
import jax.numpy as jnp
from jax import lax
from jax.experimental import pallas as pl
from jax.experimental.pallas import tpu as pltpu
from jax.experimental.pallas import tpu_sc as plsc

N = 10000
E = 320000
D = 128
R = 11

NC = 2
NS = 16
LANES = 16
NW = NC * NS

NB = 80
B = 128
EPW = NB * B
E_PAD = NW * EPW
N_PAD = 10240
RPT = N_PAD // NS


CH = 8
NCH = NB // CH


def _sc_agg_body(with_deg, table, srch, eth, dsth, *refs):
    if with_deg:
        (acc_out, deg_out, acc_sh, deg_sh, idx0, idx1, et0, et1, dst0, dst1,
         rows_a, rows_b, dvec_v, ones_v, sem_a, sem_b, ssem0, ssem1) = refs
    else:
        (acc_out, acc_sh, idx0, idx1, et0, et1, dst0, dst1,
         rows_a, rows_b, sem_a, sem_b, ssem0, ssem1) = refs
    idx = (idx0, idx1)
    et = (et0, et1)
    dst = (dst0, dst1)
    rows = (rows_a, rows_b)
    rsem = (sem_a, sem_b)
    ssem = (ssem0, ssem1)
    c = lax.axis_index("c")
    s = lax.axis_index("s")

    zv = jnp.zeros((LANES,), jnp.float32)

    @pl.loop(0, B)
    def _zero_rows(i):
        for j in range(D // LANES):
            rows_a[i, pl.ds(j * LANES, LANES)] = zv

    if with_deg:
        for j in range(RPT // LANES):
            dvec_v[pl.ds(j * LANES, LANES)] = zv
        for j in range(B // LANES):
            ones_v[pl.ds(j * LANES, LANES)] = jnp.ones((LANES,), jnp.float32)

    base = s * RPT
    for k in range(RPT // B):
        pltpu.sync_copy(rows_a, acc_sh.at[pl.ds(base + k * B, B)])
    if with_deg:
        pltpu.sync_copy(dvec_v, deg_sh.at[pl.ds(base, RPT)])

    plsc.subcore_barrier()

    def _stage(ch, p):
        sl_ch = pl.ds(ch * CH, CH)
        pltpu.async_copy(srch.at[c, s, sl_ch], idx[p], ssem[p])
        pltpu.async_copy(eth.at[c, s, sl_ch], et[p], ssem[p])
        pltpu.async_copy(dsth.at[c, s, sl_ch], dst[p], ssem[p])

    def _stage_wait(ch, p):
        sl_ch = pl.ds(ch * CH, CH)
        pltpu.make_async_copy(srch.at[c, s, sl_ch], idx[p], ssem[p]).wait()
        pltpu.make_async_copy(eth.at[c, s, sl_ch], et[p], ssem[p]).wait()
        pltpu.make_async_copy(dsth.at[c, s, sl_ch], dst[p], ssem[p]).wait()

    def _flat(p):
        for i in range(CH):
            for j in range(B // LANES):
                sl = pl.ds(j * LANES, LANES)
                idx[p][i, sl] = et[p][i, sl] * N_PAD + idx[p][i, sl]

    def _fire(g):
        chn, jn = divmod(g, CH)
        pltpu.async_copy(table.at[idx[chn % 2].at[jn]], rows[g % 2],
                         rsem[g % 2])

    _stage(0, 0)
    _stage_wait(0, 0)
    _flat(0)
    if NCH > 1:
        _stage(1, 1)
    _fire(0)
    _fire(1)
    for ch in range(NCH):
        p = ch % 2
        for j in range(CH):
            g = ch * CH + j
            rp = g % 2
            pltpu.make_async_copy(table.at[idx[p].at[j]], rows[rp],
                                  rsem[rp]).wait()
            pltpu.sync_copy(rows[rp], acc_sh.at[dst[p].at[j]], add=True)
            if j == 0 and 2 <= ch + 1 < NCH:
                _stage(ch + 1, (ch + 1) % 2)
            if j == CH - 2 and ch + 1 < NCH:
                _stage_wait(ch + 1, (ch + 1) % 2)
                _flat((ch + 1) % 2)
            if g + 2 < NB:
                _fire(g + 2)
            if with_deg:
                pltpu.sync_copy(ones_v, deg_sh.at[dst[p].at[j]], add=True)

    plsc.subcore_barrier()

    for k in range(RPT // B):
        pltpu.sync_copy(acc_sh.at[pl.ds(base + k * B, B)], rows_a)
        pltpu.sync_copy(rows_a, acc_out.at[c, pl.ds(base + k * B, B)])
    if with_deg:
        pltpu.sync_copy(deg_sh.at[pl.ds(base, RPT)], dvec_v)
        pltpu.sync_copy(dvec_v, deg_out.at[c, pl.ds(base, RPT)])


@functools.lru_cache(maxsize=None)
def _sc_agg_kernel(with_deg):
    out_type = [jax.ShapeDtypeStruct((NC, N_PAD, D), jnp.float32)]
    scratch = [pltpu.VMEM_SHARED((N_PAD, D), jnp.float32)]
    if with_deg:
        out_type.append(jax.ShapeDtypeStruct((NC, N_PAD), jnp.float32))
        scratch.append(pltpu.VMEM_SHARED((N_PAD,), jnp.float32))
    scratch += [pltpu.VMEM((CH, B), jnp.int32)] * 6
    scratch += [
        pltpu.VMEM((B, D), jnp.float32),
        pltpu.VMEM((B, D), jnp.float32),
    ]
    if with_deg:
        scratch += [
            pltpu.VMEM((RPT,), jnp.float32),
            pltpu.VMEM((B,), jnp.float32),
        ]
    scratch += [pltpu.SemaphoreType.DMA] * 4
    return pl.kernel(
        functools.partial(_sc_agg_body, with_deg),
        out_type=out_type,
        mesh=plsc.VectorSubcoreMesh(
            core_axis_name="c", subcore_axis_name="s",
            num_cores=NC, num_subcores=NS),
        scratch_types=scratch,
    )


_CB = 1024


def _scale_body(x_ref, rw_ref, o_ref):
    o_ref[...] = x_ref[...][None] * rw_ref[...][:, None, :]


def _build_table(x_pad, relw):
    out = pl.pallas_call(
        _scale_body,
        grid=(N_PAD // _CB,),
        in_specs=[
            pl.BlockSpec((_CB, D), lambda i: (i, 0)),
            pl.BlockSpec((R, D), lambda i: (0, 0)),
        ],
        out_specs=pl.BlockSpec((R, _CB, D), lambda i: (0, i, 0)),
        out_shape=jax.ShapeDtypeStruct((R, N_PAD, D), jnp.float32),
    )(x_pad, relw)
    return out.reshape(R * N_PAD, D)


def _combine_body(with_table, *refs):
    if with_table:
        acc_ref, deg_ref, x_ref, w_ref, ws_ref, b_ref, rw_ref, o_ref, t_ref = refs
    else:
        acc_ref, deg_ref, x_ref, w_ref, ws_ref, b_ref, o_ref = refs
    agg = acc_ref[0] + acc_ref[1]
    deg = deg_ref[:, 0] + deg_ref[:, 1]
    inv = 1.0 / jnp.maximum(deg, 1.0)
    agg = agg * inv[:, None]
    h = (jnp.dot(agg, w_ref[...], preferred_element_type=jnp.float32)
         + jnp.dot(x_ref[...], ws_ref[...], preferred_element_type=jnp.float32)
         + b_ref[...])
    h = jnp.maximum(h, 0.0)
    o_ref[...] = h
    if with_table:
        t_ref[...] = h[None] * rw_ref[...][:, None, :]


def _combine(acc, deg_t, x_pad, w, wself, b, relw_next=None):
    with_table = relw_next is not None
    rb = _CB if with_table else 1000
    n_out = N_PAD if with_table else N
    in_specs = [
        pl.BlockSpec((NC, rb, D), lambda i: (0, i, 0)),
        pl.BlockSpec((rb, NC), lambda i: (i, 0)),
        pl.BlockSpec((rb, D), lambda i: (i, 0)),
        pl.BlockSpec((D, D), lambda i: (0, 0)),
        pl.BlockSpec((D, D), lambda i: (0, 0)),
        pl.BlockSpec((1, D), lambda i: (0, 0)),
    ]
    out_specs = [pl.BlockSpec((rb, D), lambda i: (i, 0))]
    out_shape = [jax.ShapeDtypeStruct((n_out, D), jnp.float32)]
    args = [acc, deg_t, x_pad, w, wself, b]
    if with_table:
        in_specs.append(pl.BlockSpec((R, D), lambda i: (0, 0)))
        out_specs.append(pl.BlockSpec((R, rb, D), lambda i: (0, i, 0)))
        out_shape.append(jax.ShapeDtypeStruct((R, N_PAD, D), jnp.float32))
        args.append(relw_next)
    res = pl.pallas_call(
        functools.partial(_combine_body, with_table),
        grid=(n_out // rb,),
        in_specs=in_specs,
        out_specs=out_specs,
        out_shape=out_shape,
    )(*args)
    if with_table:
        return res[0], res[1].reshape(R * N_PAD, D)
    return res[0]


def kernel(node_init, W1, Wself1, b1, relw1, W2, Wself2, b2, relw2,
           edge_index, edge_type):
    src = edge_index[0].astype(jnp.int32)
    dst = edge_index[1].astype(jnp.int32)
    et = edge_type.astype(jnp.int32)

    pad = E_PAD - E
    pad_ar = jnp.arange(pad, dtype=jnp.int32)
    src_p = jnp.concatenate([src, (pad_ar * 97) % N])
    dst_p = jnp.concatenate([dst, N + (pad_ar % (N_PAD - N))])
    et_p = jnp.concatenate([et, pad_ar % R])
    shp = (NC, NS, NB, B)
    src_p = src_p.reshape(shp)
    dst_p = dst_p.reshape(shp)
    et_p = et_p.reshape(shp)

    b1r = b1.reshape(1, D)
    b2r = b2.reshape(1, D)
    x_pad = jnp.pad(node_init, ((0, N_PAD - N), (0, 0)))

    table1 = _build_table(x_pad, relw1)
    acc1, deg1 = _sc_agg_kernel(True)(table1, src_p, et_p, dst_p)
    deg_t = deg1.T
    h1p, table2 = _combine(acc1, deg_t, x_pad, W1, Wself1, b1r, relw2)

    (acc2,) = _sc_agg_kernel(False)(table2, src_p, et_p, dst_p)
    return _combine(acc2, deg_t, h1p, W2, Wself2, b2r)
